# Initial kernel scaffold; baseline (speedup 1.0000x reference)
import jax
import jax.numpy as jnp
from jax.experimental import pallas as pl


def kernel(x, edge_index, batch, W1, b1, p1, W2, b2, p2, W3, b3, p3, W4, b4, p4, fc1_W, fc1_b, fc2_W, fc2_b, fc3_W, fc3_b):
    raise NotImplementedError("write your pallas kernel here")


# speedup vs baseline: 1.0796x; 1.0796x over previous
import jax
import jax.numpy as jnp
from jax.experimental import pallas as pl


def _mm_body(x_ref, w_ref, o_ref):
    o_ref[...] = jnp.dot(x_ref[...], w_ref[...], preferred_element_type=jnp.float32)


def _matmul(x, w):
    return pl.pallas_call(
        _mm_body,
        out_shape=jax.ShapeDtypeStruct((x.shape[0], w.shape[1]), jnp.float32),
    )(x, w)


def _gcn_conv(x, src, dst, ew, W, b):
    n = x.shape[0]
    deg = jnp.zeros((n,), jnp.float32).at[dst].add(ew) + 1.0
    dinv = jax.lax.rsqrt(deg)
    xw = _matmul(x, W)
    norm = dinv[src] * dinv[dst] * ew
    out = jnp.zeros((n, xw.shape[1]), jnp.float32).at[dst].add(norm[:, None] * xw[src])
    out = out + (dinv * dinv)[:, None] * xw
    return out + b


def _topk_pool(x, src, dst, ew, p):
    n = x.shape[0]
    score = (x @ p) / jnp.linalg.norm(p)
    k = (n + 1) // 2
    vals, perm = jax.lax.top_k(score, k)
    x_new = x[perm] * jnp.tanh(vals)[:, None]
    kept = jnp.zeros((n,), bool).at[perm].set(True)
    new_id = jnp.zeros((n,), jnp.int32).at[perm].set(jnp.arange(k, dtype=jnp.int32))
    mask = (kept[src] & kept[dst]).astype(jnp.float32)
    return x_new, new_id[src], new_id[dst], ew * mask


def kernel(x, edge_index, batch, W1, b1, p1, W2, b2, p2, W3, b3, p3, W4, b4, p4,
           fc1_W, fc1_b, fc2_W, fc2_b, fc3_W, fc3_b):
    src = edge_index[0]
    dst = edge_index[1]
    ew = jnp.ones((src.shape[0],), jnp.float32)
    x = jax.nn.relu(_gcn_conv(x, src, dst, ew, W1, b1))
    x, src, dst, ew = _topk_pool(x, src, dst, ew, p1)
    x = jax.nn.relu(_gcn_conv(x, src, dst, ew, W2, b2))
    x, src, dst, ew = _topk_pool(x, src, dst, ew, p2)
    x = jax.nn.relu(_gcn_conv(x, src, dst, ew, W3, b3))
    x, src, dst, ew = _topk_pool(x, src, dst, ew, p3)
    x = jax.nn.relu(_gcn_conv(x, src, dst, ew, W4, b4))
    x, src, dst, ew = _topk_pool(x, src, dst, ew, p4)
    x = x.reshape(1, -1)
    x = _matmul(x, fc1_W) + fc1_b
    x = _matmul(x, fc2_W) + fc2_b
    x = _matmul(x, fc3_W) + fc3_b
    return x.reshape(-1)


# speedup vs baseline: 20.3177x; 18.8200x over previous
import functools

import jax
import jax.numpy as jnp
from jax import lax
from jax.experimental import pallas as pl
from jax.experimental.pallas import tpu as pltpu
from jax.experimental.pallas import tpu_sc as plsc

N0 = 1408
E = 90112
T = 16
EPT = E // T
C = 1408
NCHUNK = EPT // C
NPAD = 1536
DUMPMAX = NPAD - 1

LAYERS = [
    (1408, 704, 88, 1408),
    (704, 352, 48, 768),
    (352, 176, 24, 384),
    (176, 88, 16, 256),
]
RMAX = 96
NEG = -3.0e38


def _mm_body(x_ref, w_ref, o_ref):
    o_ref[...] = jnp.dot(x_ref[...], w_ref[...], preferred_element_type=jnp.float32)


def _matmul(x, w):
    return pl.pallas_call(
        _mm_body,
        out_shape=jax.ShapeDtypeStruct((x.shape[0], w.shape[1]), jnp.float32),
    )(x, w)


def _head_body(x_ref, w1_ref, b1_ref, w2_ref, b2_ref, w3_ref, b3_ref, o_ref):
    h = jnp.dot(x_ref[...], w1_ref[...], preferred_element_type=jnp.float32) + b1_ref[...]
    h = jnp.dot(h, w2_ref[...], preferred_element_type=jnp.float32) + b2_ref[...]
    o_ref[...] = jnp.dot(h, w3_ref[...], preferred_element_type=jnp.float32) + b3_ref[...]


def _head(xf, w1, b1, w2, b2, w3, b3):
    return pl.pallas_call(
        _head_body,
        out_shape=jax.ShapeDtypeStruct((1, 96), jnp.float32),
    )(xf, w1, b1, w2, b2, w3, b3)


def _rsqrt_newton(d):
    i = lax.bitcast_convert_type(d, jnp.int32)
    i = jnp.int32(0x5F3759DF) - (i >> 1)
    y = lax.bitcast_convert_type(i, jnp.float32)
    for _ in range(3):
        y = y * (1.5 - 0.5 * d * y * y)
    return y


def _sc_body(xw1, srcH, dstH, wsH, bpH,
             out,
             sX0, sX1, sXS, sAgg, sDeg, sDinv, sScore, sRank, sSrc, sDst,
             v_msg, v_one, v_esrc, v_edst, v_er1, v_er2,
             v_na, v_nb, v_z, v_d, v_sc, v_sc2, v_rk, v_scall,
             v_w, v_bp, sem):
    t = lax.axis_index("s")
    ZR = NPAD // T
    zrow = t * ZR

    def _zfill(r, c):
        v_z[r] = jnp.zeros((16,), jnp.float32)
        return c
    lax.fori_loop(0, RMAX, _zfill, 0)

    def _ofill(g, c):
        v_one[pl.ds(g * 16, 16)] = jnp.ones((16,), jnp.float32)
        return c
    lax.fori_loop(0, C // 16, _ofill, 0)
    pltpu.sync_copy(wsH, v_w)
    pltpu.sync_copy(bpH, v_bp)
    pltpu.sync_copy(srcH.at[pl.ds(t * EPT, EPT)], sSrc.at[pl.ds(t * EPT, EPT)])
    pltpu.sync_copy(dstH.at[pl.ds(t * EPT, EPT)], sDst.at[pl.ds(t * EPT, EPT)])

    xcur, xnxt = sX0, sX1
    for l, (n_in, k, rpt, NS) in enumerate(LAYERS):
        first = l == 0
        last = l == len(LAYERS) - 1
        ngrp = (rpt + 15) // 16
        row0 = t * rpt

        pltpu.sync_copy(v_z.at[pl.ds(0, ZR)], sAgg.at[pl.ds(zrow, ZR)])

        def _dzero(g, c):
            v_d[pl.ds(g * 16, 16)] = jnp.zeros((16,), jnp.float32)
            return c
        lax.fori_loop(0, RMAX // 16, _dzero, 0)
        pltpu.sync_copy(v_d, sDeg.at[pl.ds(zrow, ZR)])
        if first:
            pltpu.sync_copy(xw1.at[pl.ds(t * rpt, rpt)], xcur.at[pl.ds(t * rpt, rpt)])
        else:
            pltpu.sync_copy(xcur.at[pl.ds(row0, rpt)], v_na.at[pl.ds(0, rpt)])

            def _xw_row(r, c):
                xr = v_na[r]
                acc = jnp.zeros((16,), jnp.float32)
                for j in range(16):
                    acc = acc + xr[j] * v_w[(l - 1) * 16 + j]
                v_nb[r] = acc
                return c
            lax.fori_loop(0, rpt, _xw_row, 0)
            pltpu.sync_copy(v_nb.at[pl.ds(0, rpt)], xcur.at[pl.ds(row0, rpt)])
        plsc.subcore_barrier()

        for c in range(NCHUNK):
            eoff = t * EPT + c * C
            pltpu.sync_copy(sDst.at[pl.ds(eoff, C)], v_edst)
            pltpu.sync_copy(v_one, sDeg.at[v_edst], add=True)
        plsc.subcore_barrier()

        pltpu.sync_copy(sDeg.at[pl.ds(zrow, ZR)], v_d)
        pltpu.sync_copy(xcur.at[pl.ds(zrow, ZR)], v_na)

        def _dinv_grp(g, c):
            base = g * 16
            y = _rsqrt_newton(v_d[pl.ds(base, 16)] + 1.0)
            v_d[pl.ds(base, 16)] = y
            for ri in range(16):
                v_na[base + ri] = y[ri] * v_na[base + ri]
            return c
        lax.fori_loop(0, ZR // 16, _dinv_grp, 0)
        pltpu.sync_copy(v_d, sDinv.at[pl.ds(zrow, ZR)])
        pltpu.sync_copy(v_na, sXS.at[pl.ds(zrow, ZR)])
        plsc.subcore_barrier()

        for c in range(NCHUNK):
            eoff = t * EPT + c * C
            pltpu.sync_copy(sSrc.at[pl.ds(eoff, C)], v_esrc)
            pltpu.sync_copy(sXS.at[v_esrc], v_msg)
            pltpu.sync_copy(sDst.at[pl.ds(eoff, C)], v_edst)
            pltpu.sync_copy(v_msg, sAgg.at[v_edst], add=True)
        plsc.subcore_barrier()

        pltpu.sync_copy(sAgg.at[pl.ds(row0, rpt)], v_na.at[pl.ds(0, rpt)])
        pltpu.sync_copy(sXS.at[pl.ds(row0, rpt)], v_nb.at[pl.ds(0, rpt)])
        pltpu.sync_copy(sDinv.at[pl.ds(row0, rpt)], v_d.at[pl.ds(0, rpt)])
        bvec = v_bp[l]
        pvec = v_bp[4 + l]
        lanes = lax.iota(jnp.int32, 16)

        def _out_grp(g, c):
            base = g * 16
            dv = v_d[pl.ds(base, 16)]
            acc = jnp.full((16,), NEG, jnp.float32)
            for ri in range(16):
                o = dv[ri] * (v_na[base + ri] + v_nb[base + ri]) + bvec
                o = jnp.maximum(o, 0.0)
                v_na[base + ri] = o
                s = o * pvec
                for sh in (8, 4, 2, 1):
                    s = s + s.at[lanes ^ sh].get(mode='promise_in_bounds')
                acc = jnp.where(lanes == ri, s, acc)
            rowg = row0 + base + lanes
            v_sc[pl.ds(base, 16)] = jnp.where(rowg < n_in, acc, NEG)
            return c
        lax.fori_loop(0, ngrp, _out_grp, 0)
        pltpu.sync_copy(v_sc.at[pl.ds(0, rpt)], sScore.at[pl.ds(row0, rpt)])
        plsc.subcore_barrier()

        pltpu.sync_copy(sScore.at[pl.ds(0, NS)], v_scall.at[pl.ds(0, NS)])
        for g in range(ngrp):
            ibase = row0 + g * 16
            siv = v_scall[pl.ds(ibase, 16)]
            iota_i = ibase + lax.iota(jnp.int32, 16)

            def _rank_jg(jg, cnt):
                jbase = jg * 16
                sjv = v_scall[pl.ds(jbase, 16)]
                for jj in range(16):
                    sj = sjv[jj]
                    gt = sj > siv
                    eq = (sj == siv) & (jbase + jj < iota_i)
                    cnt = cnt + jnp.where(gt | eq, 1, 0)
                return cnt
            cnt = lax.fori_loop(0, NS // 16, _rank_jg,
                                jnp.zeros((16,), jnp.int32))
            if g * 16 + 16 > rpt:
                cnt = jnp.where(lanes < (rpt - g * 16), cnt, DUMPMAX)
            v_rk[pl.ds(g * 16, 16)] = cnt
        for g in range(ngrp, RMAX // 16):
            v_rk[pl.ds(g * 16, 16)] = jnp.full((16,), DUMPMAX, jnp.int32)
        pltpu.sync_copy(v_rk.at[pl.ds(0, rpt)], sRank.at[pl.ds(row0, rpt)])

        for g in range(RMAX // 16):
            sv = v_sc[pl.ds(g * 16, 16)]
            ev = jnp.exp(2.0 * sv)
            v_sc2[pl.ds(g * 16, 16)] = 1.0 - 2.0 / (ev + 1.0)

        def _scale_grp(g, c):
            base = g * 16
            tv = v_sc2[pl.ds(base, 16)]
            for ri in range(16):
                v_nb[base + ri] = v_na[base + ri] * tv[ri]
            return c
        lax.fori_loop(0, ngrp, _scale_grp, 0)
        pltpu.sync_copy(v_nb, xnxt.at[v_rk])
        plsc.subcore_barrier()

        if not last:
            for c in range(NCHUNK):
                eoff = t * EPT + c * C
                pltpu.sync_copy(sSrc.at[pl.ds(eoff, C)], v_esrc)
                pltpu.sync_copy(sDst.at[pl.ds(eoff, C)], v_edst)
                pltpu.sync_copy(sRank.at[v_esrc], v_er1)
                pltpu.sync_copy(sRank.at[v_edst], v_er2)

                def _remap_g(g, cc):
                    off = g * 16
                    rs = v_er1[pl.ds(off, 16)]
                    rd = v_er2[pl.ds(off, 16)]
                    live = (rs < k) & (rd < k)
                    v_esrc[pl.ds(off, 16)] = jnp.where(live, rs, k)
                    v_edst[pl.ds(off, 16)] = jnp.where(live, rd, k)
                    return cc
                lax.fori_loop(0, C // 16, _remap_g, 0)
                pltpu.sync_copy(v_esrc, sSrc.at[pl.ds(eoff, C)])
                pltpu.sync_copy(v_edst, sDst.at[pl.ds(eoff, C)])
            plsc.subcore_barrier()
        xcur, xnxt = xnxt, xcur

    @pl.when(t == 0)
    def _():
        pltpu.sync_copy(xcur.at[pl.ds(0, 88)], out)


_sc_forward = functools.partial(
    pl.kernel,
    out_type=jax.ShapeDtypeStruct((88, 16), jnp.float32),
    mesh=plsc.VectorSubcoreMesh(core_axis_name="c", subcore_axis_name="s",
                                num_cores=1),
    compiler_params=pltpu.CompilerParams(use_tc_tiling_on_sc=False),
    scratch_types=[
        pltpu.VMEM_SHARED((NPAD, 16), jnp.float32),
        pltpu.VMEM_SHARED((NPAD, 16), jnp.float32),
        pltpu.VMEM_SHARED((NPAD, 16), jnp.float32),
        pltpu.VMEM_SHARED((NPAD, 16), jnp.float32),
        pltpu.VMEM_SHARED((NPAD,), jnp.float32),
        pltpu.VMEM_SHARED((NPAD,), jnp.float32),
        pltpu.VMEM_SHARED((NPAD,), jnp.float32),
        pltpu.VMEM_SHARED((NPAD,), jnp.int32),
        pltpu.VMEM_SHARED((E,), jnp.int32),
        pltpu.VMEM_SHARED((E,), jnp.int32),
        pltpu.VMEM((C, 16), jnp.float32),
        pltpu.VMEM((C,), jnp.float32),
        pltpu.VMEM((C,), jnp.int32),
        pltpu.VMEM((C,), jnp.int32),
        pltpu.VMEM((C,), jnp.int32),
        pltpu.VMEM((C,), jnp.int32),
        pltpu.VMEM((RMAX, 16), jnp.float32),
        pltpu.VMEM((RMAX, 16), jnp.float32),
        pltpu.VMEM((RMAX, 16), jnp.float32),
        pltpu.VMEM((RMAX,), jnp.float32),
        pltpu.VMEM((RMAX,), jnp.float32),
        pltpu.VMEM((RMAX,), jnp.float32),
        pltpu.VMEM((RMAX,), jnp.int32),
        pltpu.VMEM((NPAD,), jnp.float32),
        pltpu.VMEM((48, 16), jnp.float32),
        pltpu.VMEM((8, 16), jnp.float32),
        pltpu.SemaphoreType.DMA,
    ],
)(_sc_body)


def kernel(x, edge_index, batch, W1, b1, p1, W2, b2, p2, W3, b3, p3, W4, b4, p4,
           fc1_W, fc1_b, fc2_W, fc2_b, fc3_W, fc3_b):
    src = edge_index[0]
    dst = edge_index[1]
    xw1 = _matmul(x, W1)
    Wst = jnp.concatenate([W2, W3, W4], axis=0)
    bpst = jnp.stack([
        b1, b2, b3, b4,
        p1 / jnp.linalg.norm(p1), p2 / jnp.linalg.norm(p2),
        p3 / jnp.linalg.norm(p3), p4 / jnp.linalg.norm(p4),
    ])
    x4 = _sc_forward(xw1, src, dst, Wst, bpst)
    out = _head(x4.reshape(1, N0), fc1_W, fc1_b.reshape(1, -1),
                fc2_W, fc2_b.reshape(1, -1), fc3_W, fc3_b.reshape(1, -1))
    return out.reshape(-1)


# speedup vs baseline: 20.4312x; 1.0056x over previous
import functools

import jax
import jax.numpy as jnp
from jax import lax
from jax.experimental import pallas as pl
from jax.experimental.pallas import tpu as pltpu
from jax.experimental.pallas import tpu_sc as plsc

N0 = 1408
E = 90112
T = 16
EPT = E // T
C = 2816
NCHUNK = EPT // C
NPAD = 1536
DUMPMAX = NPAD - 1

LAYERS = [
    (1408, 704, 88, 1408),
    (704, 352, 48, 768),
    (352, 176, 24, 384),
    (176, 88, 16, 256),
]
RMAX = 96
NEG = -3.0e38


def _mm_body(x_ref, w_ref, o_ref):
    o_ref[...] = jnp.dot(x_ref[...], w_ref[...], preferred_element_type=jnp.float32)


def _matmul(x, w):
    return pl.pallas_call(
        _mm_body,
        out_shape=jax.ShapeDtypeStruct((x.shape[0], w.shape[1]), jnp.float32),
    )(x, w)


def _head_body(x_ref, w1_ref, b1_ref, w2_ref, b2_ref, w3_ref, b3_ref, o_ref):
    h = jnp.dot(x_ref[...], w1_ref[...], preferred_element_type=jnp.float32) + b1_ref[...]
    h = jnp.dot(h, w2_ref[...], preferred_element_type=jnp.float32) + b2_ref[...]
    o_ref[...] = jnp.dot(h, w3_ref[...], preferred_element_type=jnp.float32) + b3_ref[...]


def _head(xf, w1, b1, w2, b2, w3, b3):
    return pl.pallas_call(
        _head_body,
        out_shape=jax.ShapeDtypeStruct((1, 96), jnp.float32),
    )(xf, w1, b1, w2, b2, w3, b3)


def _rsqrt_newton(d):
    i = lax.bitcast_convert_type(d, jnp.int32)
    i = jnp.int32(0x5F3759DF) - (i >> 1)
    y = lax.bitcast_convert_type(i, jnp.float32)
    for _ in range(3):
        y = y * (1.5 - 0.5 * d * y * y)
    return y


def _sc_body(xw1, srcH, dstH, wsH, bpH,
             out,
             sX0, sX1, sXS, sAgg, sDeg, sDinv, sScore, sRank,
             v_msg, v_one, v_src, v_dst, v_er1, v_er2,
             v_na, v_nb, v_z, v_zd, v_d, v_sc, v_sc2, v_rk, v_scall,
             v_w, v_bp, sem):
    t = lax.axis_index("s")
    ZR = NPAD // T
    zrow = t * ZR

    def _zfill(r, c):
        v_z[r] = jnp.zeros((16,), jnp.float32)
        return c
    lax.fori_loop(0, RMAX, _zfill, 0)

    def _zdfill(g, c):
        v_zd[pl.ds(g * 16, 16)] = jnp.zeros((16,), jnp.float32)
        return c
    lax.fori_loop(0, RMAX // 16, _zdfill, 0)

    def _ofill(g, c):
        v_one[pl.ds(g * 16, 16)] = jnp.ones((16,), jnp.float32)
        return c
    lax.fori_loop(0, C // 16, _ofill, 0)
    pltpu.sync_copy(wsH, v_w)
    pltpu.sync_copy(bpH, v_bp)
    for c in range(NCHUNK):
        eoff = t * EPT + c * C
        pltpu.sync_copy(srcH.at[pl.ds(eoff, C)], v_src[c])
        pltpu.sync_copy(dstH.at[pl.ds(eoff, C)], v_dst[c])
    pltpu.sync_copy(v_zd, sDeg.at[pl.ds(zrow, ZR)])

    xcur, xnxt = sX0, sX1
    for l, (n_in, k, rpt, NS) in enumerate(LAYERS):
        first = l == 0
        last = l == len(LAYERS) - 1
        ngrp = (rpt + 15) // 16
        row0 = t * rpt

        pltpu.sync_copy(v_z.at[pl.ds(0, ZR)], sAgg.at[pl.ds(zrow, ZR)])
        if first:
            pltpu.sync_copy(xw1.at[pl.ds(t * rpt, rpt)], xcur.at[pl.ds(t * rpt, rpt)])
        else:
            pltpu.sync_copy(xcur.at[pl.ds(row0, rpt)], v_na.at[pl.ds(0, rpt)])

            def _xw_row(r, c):
                xr = v_na[r]
                acc = jnp.zeros((16,), jnp.float32)
                for j in range(16):
                    acc = acc + xr[j] * v_w[(l - 1) * 16 + j]
                v_nb[r] = acc
                return c
            lax.fori_loop(0, rpt, _xw_row, 0)
            pltpu.sync_copy(v_nb.at[pl.ds(0, rpt)], xcur.at[pl.ds(row0, rpt)])
        plsc.subcore_barrier()

        if first:
            for c in range(NCHUNK):
                pltpu.sync_copy(v_one, sDeg.at[v_dst[c]], add=True)
            plsc.subcore_barrier()

        pltpu.sync_copy(sDeg.at[pl.ds(zrow, ZR)], v_d)
        pltpu.sync_copy(xcur.at[pl.ds(zrow, ZR)], v_na)

        def _dinv_grp(g, c):
            base = g * 16
            y = _rsqrt_newton(v_d[pl.ds(base, 16)] + 1.0)
            v_d[pl.ds(base, 16)] = y
            for ri in range(16):
                v_na[base + ri] = y[ri] * v_na[base + ri]
            return c
        lax.fori_loop(0, ZR // 16, _dinv_grp, 0)
        pltpu.sync_copy(v_d, sDinv.at[pl.ds(zrow, ZR)])
        pltpu.sync_copy(v_na, sXS.at[pl.ds(zrow, ZR)])
        plsc.subcore_barrier()

        for c in range(NCHUNK):
            pltpu.sync_copy(sXS.at[v_src[c]], v_msg)
            pltpu.sync_copy(v_msg, sAgg.at[v_dst[c]], add=True)
        plsc.subcore_barrier()

        pltpu.sync_copy(sAgg.at[pl.ds(row0, rpt)], v_na.at[pl.ds(0, rpt)])
        pltpu.sync_copy(sXS.at[pl.ds(row0, rpt)], v_nb.at[pl.ds(0, rpt)])
        pltpu.sync_copy(sDinv.at[pl.ds(row0, rpt)], v_d.at[pl.ds(0, rpt)])
        bvec = v_bp[l]
        pvec = v_bp[4 + l]
        lanes = lax.iota(jnp.int32, 16)

        def _out_grp(g, c):
            base = g * 16
            dv = v_d[pl.ds(base, 16)]
            acc = jnp.full((16,), NEG, jnp.float32)
            for ri in range(16):
                o = dv[ri] * (v_na[base + ri] + v_nb[base + ri]) + bvec
                o = jnp.maximum(o, 0.0)
                v_na[base + ri] = o
                s = o * pvec
                for sh in (8, 4, 2, 1):
                    s = s + s.at[lanes ^ sh].get(mode='promise_in_bounds')
                acc = jnp.where(lanes == ri, s, acc)
            rowg = row0 + base + lanes
            v_sc[pl.ds(base, 16)] = jnp.where(rowg < n_in, acc, NEG)
            return c
        lax.fori_loop(0, ngrp, _out_grp, 0)
        pltpu.sync_copy(v_sc.at[pl.ds(0, rpt)], sScore.at[pl.ds(row0, rpt)])
        plsc.subcore_barrier()

        pltpu.sync_copy(sScore.at[pl.ds(0, NS)], v_scall.at[pl.ds(0, NS)])
        for g in range(ngrp):
            ibase = row0 + g * 16
            siv = v_scall[pl.ds(ibase, 16)]
            iota_i = ibase + lax.iota(jnp.int32, 16)

            def _rank_jg(jg, cnt):
                jbase = jg * 16
                sjv = v_scall[pl.ds(jbase, 16)]
                for jj in range(16):
                    sj = sjv[jj]
                    gt = sj > siv
                    eq = (sj == siv) & (jbase + jj < iota_i)
                    cnt = cnt + jnp.where(gt | eq, 1, 0)
                return cnt
            cnt = lax.fori_loop(0, NS // 16, _rank_jg,
                                jnp.zeros((16,), jnp.int32))
            if g * 16 + 16 > rpt:
                cnt = jnp.where(lanes < (rpt - g * 16), cnt, DUMPMAX)
            v_rk[pl.ds(g * 16, 16)] = cnt
        for g in range(ngrp, RMAX // 16):
            v_rk[pl.ds(g * 16, 16)] = jnp.full((16,), DUMPMAX, jnp.int32)
        pltpu.sync_copy(v_rk.at[pl.ds(0, rpt)], sRank.at[pl.ds(row0, rpt)])

        for g in range(RMAX // 16):
            sv = v_sc[pl.ds(g * 16, 16)]
            ev = jnp.exp(2.0 * sv)
            v_sc2[pl.ds(g * 16, 16)] = 1.0 - 2.0 / (ev + 1.0)

        def _scale_grp(g, c):
            base = g * 16
            tv = v_sc2[pl.ds(base, 16)]
            for ri in range(16):
                v_nb[base + ri] = v_na[base + ri] * tv[ri]
            return c
        lax.fori_loop(0, ngrp, _scale_grp, 0)
        pltpu.sync_copy(v_nb, xnxt.at[v_rk])
        if not last:
            pltpu.sync_copy(v_zd, sDeg.at[pl.ds(zrow, ZR)])
        plsc.subcore_barrier()

        if not last:
            for c in range(NCHUNK):
                pltpu.sync_copy(sRank.at[v_src[c]], v_er1)
                pltpu.sync_copy(sRank.at[v_dst[c]], v_er2)

                def _remap_g(g, cc):
                    off = g * 16
                    rs = v_er1[pl.ds(off, 16)]
                    rd = v_er2[pl.ds(off, 16)]
                    live = (rs < k) & (rd < k)
                    v_src[c][pl.ds(off, 16)] = jnp.where(live, rs, k)
                    v_dst[c][pl.ds(off, 16)] = jnp.where(live, rd, k)
                    return cc
                lax.fori_loop(0, C // 16, _remap_g, 0)
                pltpu.sync_copy(v_one, sDeg.at[v_dst[c]], add=True)
            plsc.subcore_barrier()
        xcur, xnxt = xnxt, xcur

    @pl.when(t == 0)
    def _():
        pltpu.sync_copy(xcur.at[pl.ds(0, 88)], out)


def _sc_wrap(xw1, srcH, dstH, wsH, bpH, out,
             sX0, sX1, sXS, sAgg, sDeg, sDinv, sScore, sRank,
             v_msg, v_one, v_srcA, v_srcB, v_dstA, v_dstB, v_er1, v_er2,
             v_na, v_nb, v_z, v_zd, v_d, v_sc, v_sc2, v_rk, v_scall,
             v_w, v_bp, sem):
    _sc_body(xw1, srcH, dstH, wsH, bpH, out,
             sX0, sX1, sXS, sAgg, sDeg, sDinv, sScore, sRank,
             v_msg, v_one, [v_srcA, v_srcB], [v_dstA, v_dstB], v_er1, v_er2,
             v_na, v_nb, v_z, v_zd, v_d, v_sc, v_sc2, v_rk, v_scall,
             v_w, v_bp, sem)


_sc_forward = functools.partial(
    pl.kernel,
    out_type=jax.ShapeDtypeStruct((88, 16), jnp.float32),
    mesh=plsc.VectorSubcoreMesh(core_axis_name="c", subcore_axis_name="s",
                                num_cores=1),
    compiler_params=pltpu.CompilerParams(use_tc_tiling_on_sc=False),
    scratch_types=[
        pltpu.VMEM_SHARED((NPAD, 16), jnp.float32),
        pltpu.VMEM_SHARED((NPAD, 16), jnp.float32),
        pltpu.VMEM_SHARED((NPAD, 16), jnp.float32),
        pltpu.VMEM_SHARED((NPAD, 16), jnp.float32),
        pltpu.VMEM_SHARED((NPAD,), jnp.float32),
        pltpu.VMEM_SHARED((NPAD,), jnp.float32),
        pltpu.VMEM_SHARED((NPAD,), jnp.float32),
        pltpu.VMEM_SHARED((NPAD,), jnp.int32),
        pltpu.VMEM((C, 16), jnp.float32),
        pltpu.VMEM((C,), jnp.float32),
        pltpu.VMEM((C,), jnp.int32),
        pltpu.VMEM((C,), jnp.int32),
        pltpu.VMEM((C,), jnp.int32),
        pltpu.VMEM((C,), jnp.int32),
        pltpu.VMEM((C,), jnp.int32),
        pltpu.VMEM((C,), jnp.int32),
        pltpu.VMEM((RMAX, 16), jnp.float32),
        pltpu.VMEM((RMAX, 16), jnp.float32),
        pltpu.VMEM((RMAX, 16), jnp.float32),
        pltpu.VMEM((RMAX,), jnp.float32),
        pltpu.VMEM((RMAX,), jnp.float32),
        pltpu.VMEM((RMAX,), jnp.float32),
        pltpu.VMEM((RMAX,), jnp.float32),
        pltpu.VMEM((RMAX,), jnp.int32),
        pltpu.VMEM((NPAD,), jnp.float32),
        pltpu.VMEM((48, 16), jnp.float32),
        pltpu.VMEM((8, 16), jnp.float32),
        pltpu.SemaphoreType.DMA,
    ],
)(_sc_wrap)


def kernel(x, edge_index, batch, W1, b1, p1, W2, b2, p2, W3, b3, p3, W4, b4, p4,
           fc1_W, fc1_b, fc2_W, fc2_b, fc3_W, fc3_b):
    src = edge_index[0]
    dst = edge_index[1]
    xw1 = _matmul(x, W1)
    Wst = jnp.concatenate([W2, W3, W4], axis=0)
    bpst = jnp.stack([
        b1, b2, b3, b4,
        p1 / jnp.linalg.norm(p1), p2 / jnp.linalg.norm(p2),
        p3 / jnp.linalg.norm(p3), p4 / jnp.linalg.norm(p4),
    ])
    x4 = _sc_forward(xw1, src, dst, Wst, bpst)
    out = _head(x4.reshape(1, N0), fc1_W, fc1_b.reshape(1, -1),
                fc2_W, fc2_b.reshape(1, -1), fc3_W, fc3_b.reshape(1, -1))
    return out.reshape(-1)


# speedup vs baseline: 81.6019x; 3.9940x over previous
import functools

import jax
import jax.numpy as jnp
from jax import lax
from jax.experimental import pallas as pl
from jax.experimental.pallas import tpu as pltpu
from jax.experimental.pallas import tpu_sc as plsc

N0 = 1408
E = 90112
T = 16
EPT = E // T
Q = 704
NQ = EPT // Q
STAGE = EPT + Q
NPAD = 1536
DUMPMAX = NPAD - 1

LAYERS = [
    (1408, 704, 88, 1408),
    (704, 352, 48, 768),
    (352, 176, 24, 384),
    (176, 88, 16, 256),
]
RMAX = 96
NEG = -3.0e38


def _mm_body(x_ref, w_ref, o_ref):
    o_ref[...] = jnp.dot(x_ref[...], w_ref[...], preferred_element_type=jnp.float32)


def _matmul(x, w):
    return pl.pallas_call(
        _mm_body,
        out_shape=jax.ShapeDtypeStruct((x.shape[0], w.shape[1]), jnp.float32),
    )(x, w)


def _head_body(x_ref, w1_ref, b1_ref, w2_ref, b2_ref, w3_ref, b3_ref, o_ref):
    h = jnp.dot(x_ref[...], w1_ref[...], preferred_element_type=jnp.float32) + b1_ref[...]
    h = jnp.dot(h, w2_ref[...], preferred_element_type=jnp.float32) + b2_ref[...]
    o_ref[...] = jnp.dot(h, w3_ref[...], preferred_element_type=jnp.float32) + b3_ref[...]


def _head(xf, w1, b1, w2, b2, w3, b3):
    return pl.pallas_call(
        _head_body,
        out_shape=jax.ShapeDtypeStruct((1, 96), jnp.float32),
    )(xf, w1, b1, w2, b2, w3, b3)


def _rsqrt_newton(d):
    i = lax.bitcast_convert_type(d, jnp.int32)
    i = jnp.int32(0x5F3759DF) - (i >> 1)
    y = lax.bitcast_convert_type(i, jnp.float32)
    for _ in range(3):
        y = y * (1.5 - 0.5 * d * y * y)
    return y


def _sc_body(xw1, srcH, dstH, wsH, bpH,
             out,
             sX0, sX1, sXS, sAgg, sDeg, sDinv, sScore, sRank,
             v_msg, v_one, v_src1, v_dst2, v_stD, v_rt,
             v_na, v_nb, v_z, v_zd, v_d, v_sc, v_sc2, v_rk, v_scall,
             v_w, v_bp, m_ref, sem):
    t = lax.axis_index("s")
    ZR = NPAD // T
    zrow = t * ZR

    def _zfill(r, c):
        v_z[r] = jnp.zeros((16,), jnp.float32)
        return c
    lax.fori_loop(0, RMAX, _zfill, 0)

    def _zdfill(g, c):
        v_zd[pl.ds(g * 16, 16)] = jnp.zeros((16,), jnp.float32)
        return c
    lax.fori_loop(0, RMAX // 16, _zdfill, 0)

    def _ofill(g, c):
        v_one[pl.ds(g * 16, 16)] = jnp.ones((16,), jnp.float32)
        return c
    lax.fori_loop(0, Q // 16, _ofill, 0)
    pltpu.sync_copy(wsH, v_w)
    pltpu.sync_copy(bpH, v_bp)
    pltpu.sync_copy(srcH.at[pl.ds(t * EPT, EPT)], v_src1.at[pl.ds(0, EPT)])
    for q in range(NQ):
        pltpu.sync_copy(dstH.at[pl.ds(t * EPT + q * Q, Q)], v_dst2.at[q])
    pltpu.sync_copy(v_zd, sDeg.at[pl.ds(zrow, ZR)])

    xcur, xnxt = sX0, sX1
    for l, (n_in, k, rpt, NS) in enumerate(LAYERS):
        first = l == 0
        last = l == len(LAYERS) - 1
        ngrp = (rpt + 15) // 16
        row0 = t * rpt
        if first:
            m = EPT
            nq = NQ
        else:
            m = m_ref[0]
            nq = (m + (Q - 1)) // Q

        pltpu.sync_copy(v_z.at[pl.ds(0, ZR)], sAgg.at[pl.ds(zrow, ZR)])
        if first:
            pltpu.sync_copy(xw1.at[pl.ds(t * rpt, rpt)], xcur.at[pl.ds(t * rpt, rpt)])
        else:
            pltpu.sync_copy(xcur.at[pl.ds(row0, rpt)], v_na.at[pl.ds(0, rpt)])

            def _xw_row(r, c):
                xr = v_na[r]
                acc = jnp.zeros((16,), jnp.float32)
                for j in range(16):
                    acc = acc + xr[j] * v_w[(l - 1) * 16 + j]
                v_nb[r] = acc
                return c
            lax.fori_loop(0, rpt, _xw_row, 0)
            pltpu.sync_copy(v_nb.at[pl.ds(0, rpt)], xcur.at[pl.ds(row0, rpt)])
        plsc.subcore_barrier()

        if first:
            for q in range(NQ):
                pltpu.sync_copy(v_one, sDeg.at[v_dst2.at[q]], add=True)
            plsc.subcore_barrier()

        pltpu.sync_copy(sDeg.at[pl.ds(zrow, ZR)], v_d)
        pltpu.sync_copy(xcur.at[pl.ds(zrow, ZR)], v_na)

        def _dinv_grp(g, c):
            base = g * 16
            y = _rsqrt_newton(v_d[pl.ds(base, 16)] + 1.0)
            v_d[pl.ds(base, 16)] = y
            for ri in range(16):
                v_na[base + ri] = y[ri] * v_na[base + ri]
            return c
        lax.fori_loop(0, ZR // 16, _dinv_grp, 0)
        pltpu.sync_copy(v_d, sDinv.at[pl.ds(zrow, ZR)])
        pltpu.sync_copy(v_na, sXS.at[pl.ds(zrow, ZR)])
        plsc.subcore_barrier()

        def _msg_q(q, c):
            pltpu.sync_copy(sXS.at[v_src1.at[pl.ds(q * Q, Q)]], v_msg)
            pltpu.sync_copy(v_msg, sAgg.at[v_dst2.at[q]], add=True)
            return c
        if first:
            for q in range(NQ):
                _msg_q(q, 0)
        else:
            lax.fori_loop(0, nq, _msg_q, 0)
        plsc.subcore_barrier()

        pltpu.sync_copy(sAgg.at[pl.ds(row0, rpt)], v_na.at[pl.ds(0, rpt)])
        pltpu.sync_copy(sXS.at[pl.ds(row0, rpt)], v_nb.at[pl.ds(0, rpt)])
        pltpu.sync_copy(sDinv.at[pl.ds(row0, rpt)], v_d.at[pl.ds(0, rpt)])
        bvec = v_bp[l]
        pvec = v_bp[4 + l]
        lanes = lax.iota(jnp.int32, 16)

        def _out_grp(g, c):
            base = g * 16
            dv = v_d[pl.ds(base, 16)]
            acc = jnp.full((16,), NEG, jnp.float32)
            for ri in range(16):
                o = dv[ri] * (v_na[base + ri] + v_nb[base + ri]) + bvec
                o = jnp.maximum(o, 0.0)
                v_na[base + ri] = o
                s = o * pvec
                for sh in (8, 4, 2, 1):
                    s = s + s.at[lanes ^ sh].get(mode='promise_in_bounds')
                acc = jnp.where(lanes == ri, s, acc)
            rowg = row0 + base + lanes
            v_sc[pl.ds(base, 16)] = jnp.where(rowg < n_in, acc, NEG)
            return c
        lax.fori_loop(0, ngrp, _out_grp, 0)
        pltpu.sync_copy(v_sc.at[pl.ds(0, rpt)], sScore.at[pl.ds(row0, rpt)])
        plsc.subcore_barrier()

        pltpu.sync_copy(sScore.at[pl.ds(0, NS)], v_scall.at[pl.ds(0, NS)])
        for g in range(ngrp):
            ibase = row0 + g * 16
            siv = v_scall[pl.ds(ibase, 16)]
            iota_i = ibase + lax.iota(jnp.int32, 16)

            def _rank_jg(jg, cnt):
                jbase = jg * 16
                sjv = v_scall[pl.ds(jbase, 16)]
                for jj in range(16):
                    sj = sjv[jj]
                    gt = sj > siv
                    eq = (sj == siv) & (jbase + jj < iota_i)
                    cnt = cnt + jnp.where(gt | eq, 1, 0)
                return cnt
            cnt = lax.fori_loop(0, NS // 16, _rank_jg,
                                jnp.zeros((16,), jnp.int32))
            if g * 16 + 16 > rpt:
                cnt = jnp.where(lanes < (rpt - g * 16), cnt, DUMPMAX)
            v_rk[pl.ds(g * 16, 16)] = cnt
        for g in range(ngrp, RMAX // 16):
            v_rk[pl.ds(g * 16, 16)] = jnp.full((16,), DUMPMAX, jnp.int32)
        pltpu.sync_copy(v_rk.at[pl.ds(0, rpt)], sRank.at[pl.ds(row0, rpt)])

        for g in range(RMAX // 16):
            sv = v_sc[pl.ds(g * 16, 16)]
            ev = jnp.exp(2.0 * sv)
            v_sc2[pl.ds(g * 16, 16)] = 1.0 - 2.0 / (ev + 1.0)

        def _scale_grp(g, c):
            base = g * 16
            tv = v_sc2[pl.ds(base, 16)]
            for ri in range(16):
                v_nb[base + ri] = v_na[base + ri] * tv[ri]
            return c
        lax.fori_loop(0, ngrp, _scale_grp, 0)
        pltpu.sync_copy(v_nb, xnxt.at[v_rk])
        if not last:
            pltpu.sync_copy(v_zd, sDeg.at[pl.ds(zrow, ZR)])
        plsc.subcore_barrier()

        if not last:
            pltpu.sync_copy(sRank.at[pl.ds(0, NS)], v_rt.at[pl.ds(0, NS)])
            ngq = (m + 15) // 16
            GPR = Q // 16

            def _cmp_g(g, mm):
                off = g * 16
                sv = v_src1[pl.ds(off, 16)]
                dr = g // GPR
                dc = (g % GPR) * 16
                dv = v_dst2[dr, pl.ds(dc, 16)]
                rs = plsc.load_gather(v_rt, [sv])
                rd = plsc.load_gather(v_rt, [dv])
                live = (rs < k) & (rd < k) & (off + lanes < m)
                key = jnp.where(live, 0, 1).astype(jnp.uint32)
                _, rs2 = plsc.sort_key_val(key, rs)
                _, rd2 = plsc.sort_key_val(key, rd)
                v_src1[pl.ds(mm, 16)] = rs2
                v_stD[pl.ds(mm, 16)] = rd2
                pc = plsc.all_reduce_population_count(live)
                return mm + pc[0]
            mm = lax.fori_loop(0, ngq, _cmp_g, jnp.int32(0))
            for g in range(Q // 16):
                v_src1[pl.ds(mm + g * 16, 16)] = jnp.full((16,), k, jnp.int32)
                v_stD[pl.ds(mm + g * 16, 16)] = jnp.full((16,), k, jnp.int32)
            nq2 = (mm + (Q - 1)) // Q

            def _cp_q(q, c):
                for gg in range(Q // 16):
                    v_dst2[q, pl.ds(gg * 16, 16)] = v_stD[pl.ds(q * Q + gg * 16, 16)]
                pltpu.sync_copy(v_one, sDeg.at[v_dst2.at[q]], add=True)
                return c
            lax.fori_loop(0, nq2, _cp_q, 0)
            m_ref[0] = mm
            plsc.subcore_barrier()
        xcur, xnxt = xnxt, xcur

    @pl.when(t == 0)
    def _():
        pltpu.sync_copy(xcur.at[pl.ds(0, 88)], out)


_sc_forward = functools.partial(
    pl.kernel,
    out_type=jax.ShapeDtypeStruct((88, 16), jnp.float32),
    mesh=plsc.VectorSubcoreMesh(core_axis_name="c", subcore_axis_name="s",
                                num_cores=1),
    compiler_params=pltpu.CompilerParams(use_tc_tiling_on_sc=False,
                                         needs_layout_passes=False),
    scratch_types=[
        pltpu.VMEM_SHARED((NPAD, 16), jnp.float32),
        pltpu.VMEM_SHARED((NPAD, 16), jnp.float32),
        pltpu.VMEM_SHARED((NPAD, 16), jnp.float32),
        pltpu.VMEM_SHARED((NPAD, 16), jnp.float32),
        pltpu.VMEM_SHARED((NPAD,), jnp.float32),
        pltpu.VMEM_SHARED((NPAD,), jnp.float32),
        pltpu.VMEM_SHARED((NPAD,), jnp.float32),
        pltpu.VMEM_SHARED((NPAD,), jnp.int32),
        pltpu.VMEM((Q, 16), jnp.float32),
        pltpu.VMEM((Q,), jnp.float32),
        pltpu.VMEM((STAGE,), jnp.int32),
        pltpu.VMEM((NQ, Q), jnp.int32),
        pltpu.VMEM((STAGE,), jnp.int32),
        pltpu.VMEM((EPT,), jnp.int32),
        pltpu.VMEM((RMAX, 16), jnp.float32),
        pltpu.VMEM((RMAX, 16), jnp.float32),
        pltpu.VMEM((RMAX, 16), jnp.float32),
        pltpu.VMEM((RMAX,), jnp.float32),
        pltpu.VMEM((RMAX,), jnp.float32),
        pltpu.VMEM((RMAX,), jnp.float32),
        pltpu.VMEM((RMAX,), jnp.float32),
        pltpu.VMEM((RMAX,), jnp.int32),
        pltpu.VMEM((NPAD,), jnp.float32),
        pltpu.VMEM((48, 16), jnp.float32),
        pltpu.VMEM((8, 16), jnp.float32),
        pltpu.SMEM((8,), jnp.int32),
        pltpu.SemaphoreType.DMA,
    ],
)(_sc_body)


def kernel(x, edge_index, batch, W1, b1, p1, W2, b2, p2, W3, b3, p3, W4, b4, p4,
           fc1_W, fc1_b, fc2_W, fc2_b, fc3_W, fc3_b):
    src = edge_index[0]
    dst = edge_index[1]
    xw1 = _matmul(x, W1)
    Wst = jnp.concatenate([W2, W3, W4], axis=0)
    bpst = jnp.stack([
        b1, b2, b3, b4,
        p1 / jnp.linalg.norm(p1), p2 / jnp.linalg.norm(p2),
        p3 / jnp.linalg.norm(p3), p4 / jnp.linalg.norm(p4),
    ])
    x4 = _sc_forward(xw1, src, dst, Wst, bpst)
    out = _head(x4.reshape(1, N0), fc1_W, fc1_b.reshape(1, -1),
                fc2_W, fc2_b.reshape(1, -1), fc3_W, fc3_b.reshape(1, -1))
    return out.reshape(-1)
